# baseline (speedup 1.0000x reference)
import functools

import jax
import jax.numpy as jnp
from jax import lax
from jax.experimental import pallas as pl
from jax.experimental.pallas import tpu as pltpu
from jax.experimental.pallas import tpu_sc as plsc

B = 1024
K = 16
D = 32
NUM_REL = 32
NUM_ENT = 100000
NUM_USR = 10000

NC = 2
NS = 16
NW = NC * NS
BPW = B // NW
NG = BPW // 16


def _sc_fused_kernel():
  mesh = plsc.VectorSubcoreMesh(
      core_axis_name="c", subcore_axis_name="s",
      num_cores=NC, num_subcores=NS)

  @functools.partial(
      pl.kernel,
      mesh=mesh,
      compiler_params=pltpu.CompilerParams(use_tc_tiling_on_sc=True,
                                           needs_layout_passes=False),
      out_type=jax.ShapeDtypeStruct((B,), jnp.float32),
      scratch_types=[
          pltpu.VMEM((BPW,), jnp.int32),
          pltpu.VMEM((BPW,), jnp.int32),
          pltpu.VMEM((BPW, 128), jnp.int32),
          pltpu.VMEM((BPW, 128), jnp.int32),
          pltpu.VMEM((BPW, 128), jnp.float32),
          pltpu.VMEM((BPW, 128), jnp.float32),
          pltpu.VMEM((BPW * K, 128), jnp.float32),
          pltpu.VMEM((D, 16), jnp.float32),
          pltpu.VMEM((D, 16), jnp.float32),
          pltpu.VMEM((BPW,), jnp.float32),
          pltpu.SemaphoreType.DMA,
          pltpu.SemaphoreType.DMA,
      ],
  )
  def sc_fused(u_h, v_h, ae_h, ar_h, usr_h, ent_h, relT_h, wT_h, b_h,
               out_h, uix, vix, aeb, arb, ueb, vsb, neb, uet, xt, res,
               sem_r, sem_a):
    wid = lax.axis_index("s") * NC + lax.axis_index("c")
    base = wid * BPW
    pltpu.sync_copy(u_h.at[pl.ds(base, BPW)], uix)
    pltpu.sync_copy(v_h.at[pl.ds(base, BPW)], vix)
    def fetch_rows(c, carry):
      uvec = uix[pl.ds(c * 16, 16)]
      vvec = vix[pl.ds(c * 16, 16)]
      for l in range(16):
        j = c * 16 + l
        vv = vvec[l]
        uu = uvec[l]
        vg = vv >> 3
        pltpu.async_copy(ae_h.at[vg], aeb.at[j], sem_a)
        pltpu.async_copy(ar_h.at[vg], arb.at[j], sem_a)
        pltpu.async_copy(usr_h.at[uu >> 2], ueb.at[j], sem_r)
        pltpu.async_copy(ent_h.at[vv >> 2], vsb.at[j], sem_r)
      return carry
    lax.fori_loop(0, NG, fetch_rows, 0)
    def drain_ae(j, carry):
      pltpu.make_async_copy(ae_h.at[0], aeb.at[j], sem_a).wait()
      return carry
    lax.fori_loop(0, BPW, drain_ae, 0)
    def fetch_ne(c, carry):
      vvec = vix[pl.ds(c * 16, 16)]
      for l in range(16):
        j = c * 16 + l
        off = (vvec[l] & 7) * K
        row16 = aeb[j, pl.ds(off, 16)]
        for k in range(K):
          e = row16[k]
          pltpu.async_copy(ent_h.at[e >> 2], neb.at[j * K + k], sem_r)
      return carry
    lax.fori_loop(0, NG, fetch_ne, 0)
    def drain_rest(j, carry):
      pltpu.make_async_copy(ar_h.at[0], arb.at[j], sem_a).wait()
      pltpu.make_async_copy(usr_h.at[0], ueb.at[j], sem_r).wait()
      pltpu.make_async_copy(ent_h.at[0], vsb.at[j], sem_r).wait()
      for k in range(K):
        pltpu.make_async_copy(ent_h.at[0], neb.at[j * K + k], sem_r).wait()
      return carry
    lax.fori_loop(0, BPW, drain_rest, 0)
    pl.run_scoped(
        lambda relT_v, wT_v, b_v: _sc_math(
            uix, vix, aeb, arb, ueb, vsb, neb, uet, xt, res,
            relT_v, wT_v, b_v, relT_h, wT_h, b_h, out_h, base),
        pltpu.VMEM((D, NUM_REL), jnp.float32),
        pltpu.VMEM((D, D), jnp.float32),
        pltpu.VMEM((D,), jnp.float32),
    )

  def _sc_math(uix, vix, aeb, arb, ueb, vsb, neb, uet, xt, res,
               relT_v, wT_v, b_v, relT_h, wT_h, b_h, out_h, base):
    pltpu.sync_copy(relT_h, relT_v)
    pltpu.sync_copy(wT_h, wT_v)
    pltpu.sync_copy(b_h, b_v)
    lanes = lax.iota(jnp.int32, 16)
    for g in range(NG):
      bvec = lanes + g * 16
      uvec = uix[pl.ds(g * 16, 16)]
      vvec = vix[pl.ds(g * 16, 16)]
      uoff = (uvec & 3) * D
      voff = (vvec & 3) * D
      aoff = (vvec & 7) * K
      relids = [plsc.load_gather(arb, [bvec, aoff + k]) for k in range(K)]
      def logits_step(d, s):
        dvec = jnp.full((16,), 0, jnp.int32) + d
        ued = plsc.load_gather(ueb, [bvec, uoff + d])
        uet[d] = ued
        return tuple(
            s[k] + ued * plsc.load_gather(relT_v, [dvec, relids[k]])
            for k in range(K))
      s = lax.fori_loop(
          0, D, logits_step,
          tuple(jnp.zeros((16,), jnp.float32) for _ in range(K)))
      m = s[0]
      for k in range(1, K):
        m = jnp.maximum(m, s[k])
      es = [jnp.exp(s[k] - m) for k in range(K)]
      tot = es[0]
      for k in range(1, K):
        tot = tot + es[k]
      inv = 1.0 / tot
      p = [es[k] * inv for k in range(K)]
      nrows = [bvec * K + k for k in range(K)]
      nbrv = [plsc.load_gather(aeb, [bvec, aoff + k]) for k in range(K)]
      noffs = [(nbrv[k] & 3) * D for k in range(K)]
      def wsum_step(d, carry):
        acc = plsc.load_gather(vsb, [bvec, voff + d])
        for k in range(K):
          ned = plsc.load_gather(neb, [nrows[k], noffs[k] + d])
          acc = acc + p[k] * ned
        xt[d] = acc
        return carry
      lax.fori_loop(0, D, wsum_step, 0)
      def lin_step(do, y):
        dovec = jnp.full((16,), 0, jnp.int32) + do
        accw = plsc.load_gather(b_v, [dovec])
        for j in range(D):
          wv = plsc.load_gather(wT_v, [jnp.full((16,), j, jnp.int32), dovec])
          accw = accw + xt[j] * wv
        vu = jnp.maximum(accw, 0.0)
        return y + plsc.load_gather(uet, [jnp.full((16,), 0, jnp.int32) + do,
                                          lanes]) * vu
      y = lax.fori_loop(0, D, lin_step, jnp.zeros((16,), jnp.float32))
      res[pl.ds(g * 16, 16)] = 1.0 / (1.0 + jnp.exp(-y))
    pltpu.sync_copy(res, out_h.at[pl.ds(base, BPW)])

  return sc_fused


def kernel(u, v, adj_ent, adj_rel, usr_emb, ent_emb, rel_emb, W, b):
  ae = adj_ent.astype(jnp.int32).reshape(NUM_ENT // 8, 128)
  ar = adj_rel.astype(jnp.int32).reshape(NUM_ENT // 8, 128)
  usr = usr_emb.reshape(NUM_USR // 4, 128)
  ent = ent_emb.reshape(NUM_ENT // 4, 128)
  out = _sc_fused_kernel()(
      u.astype(jnp.int32), v.astype(jnp.int32), ae, ar, usr, ent,
      rel_emb.T, W.T, b)
  return out


# speedup vs baseline: 1.7899x; 1.7899x over previous
import functools

import jax
import jax.numpy as jnp
from jax import lax
from jax.experimental import pallas as pl
from jax.experimental.pallas import tpu as pltpu
from jax.experimental.pallas import tpu_sc as plsc

B = 1024
K = 16
D = 32
NUM_REL = 32
NUM_ENT = 100000
NUM_USR = 10000

NC = 2
NS = 16
NW = NC * NS
BPW = B // NW
NG = BPW // 16
TPW = BPW // 8


def _sc_fused_kernel():
  mesh = plsc.VectorSubcoreMesh(
      core_axis_name="c", subcore_axis_name="s",
      num_cores=NC, num_subcores=NS)

  @functools.partial(
      pl.kernel,
      mesh=mesh,
      compiler_params=pltpu.CompilerParams(use_tc_tiling_on_sc=True,
                                           needs_layout_passes=False),
      out_type=jax.ShapeDtypeStruct((B,), jnp.float32),
      scratch_types=[
          pltpu.VMEM((BPW,), jnp.int32),
          pltpu.VMEM((BPW,), jnp.int32),
          pltpu.VMEM((TPW, 8, K), jnp.int32),
          pltpu.VMEM((TPW, 8, K), jnp.int32),
          pltpu.VMEM((TPW, 8, D), jnp.float32),
          pltpu.VMEM((TPW, 8, D), jnp.float32),
          pltpu.VMEM((BPW * K // 8, 8, D), jnp.float32),
          pltpu.VMEM((D, 16), jnp.float32),
          pltpu.VMEM((D, 16), jnp.float32),
          pltpu.VMEM((BPW,), jnp.float32),
          pltpu.SemaphoreType.DMA,
          pltpu.SemaphoreType.DMA,
      ],
  )
  def sc_fused(u_h, v_h, ae_h, ar_h, usr_h, ent_h, relT_h, wT_h, b_h,
               out_h, uix, vix, aeb, arb, ueb, vsb, neb, uet, xt, res,
               sem_r, sem_a):
    wid = lax.axis_index("s") * NC + lax.axis_index("c")
    base = wid * BPW
    pltpu.sync_copy(u_h.at[pl.ds(base, BPW)], uix)
    pltpu.sync_copy(v_h.at[pl.ds(base, BPW)], vix)
    def fetch_rows(c, carry):
      uvec = uix[pl.ds(c * 16, 16)]
      vvec = vix[pl.ds(c * 16, 16)]
      for l in range(16):
        j = c * 16 + l
        tj, sj = j >> 3, j & 7
        vv = vvec[l]
        uu = uvec[l]
        vt, vs2 = vv >> 3, vv & 7
        pltpu.async_copy(ae_h.at[vt, vs2], aeb.at[tj, sj], sem_a)
        pltpu.async_copy(ar_h.at[vt, vs2], arb.at[tj, sj], sem_a)
        pltpu.async_copy(usr_h.at[uu >> 3, uu & 7], ueb.at[tj, sj], sem_r)
        pltpu.async_copy(ent_h.at[vt, vs2], vsb.at[tj, sj], sem_r)
      return carry
    lax.fori_loop(0, NG, fetch_rows, 0)
    def drain_ae(j, carry):
      pltpu.make_async_copy(ae_h.at[0, 0], aeb.at[j >> 3, j & 7],
                            sem_a).wait()
      return carry
    lax.fori_loop(0, BPW, drain_ae, 0)
    def fetch_ne(c, carry):
      for l in range(16):
        j = c * 16 + l
        row16 = aeb[j >> 3, j & 7]
        for k in range(K):
          e = row16[k]
          r = j * K + k
          pltpu.async_copy(ent_h.at[e >> 3, e & 7],
                           neb.at[r >> 3, r & 7], sem_r)
      return carry
    lax.fori_loop(0, NG, fetch_ne, 0)
    def drain_rest(j, carry):
      tj, sj = j >> 3, j & 7
      pltpu.make_async_copy(ar_h.at[0, 0], arb.at[tj, sj], sem_a).wait()
      pltpu.make_async_copy(usr_h.at[0, 0], ueb.at[tj, sj], sem_r).wait()
      pltpu.make_async_copy(ent_h.at[0, 0], vsb.at[tj, sj], sem_r).wait()
      for k in range(K):
        r = j * K + k
        pltpu.make_async_copy(ent_h.at[0, 0], neb.at[r >> 3, r & 7],
                              sem_r).wait()
      return carry
    lax.fori_loop(0, BPW, drain_rest, 0)
    pl.run_scoped(
        lambda relT_v, wT_v, b_v: _sc_math(
            aeb, arb, ueb, vsb, neb, uet, xt, res,
            relT_v, wT_v, b_v, relT_h, wT_h, b_h, out_h, base),
        pltpu.VMEM((D, NUM_REL), jnp.float32),
        pltpu.VMEM((D, D), jnp.float32),
        pltpu.VMEM((D,), jnp.float32),
    )

  def _sc_math(aeb, arb, ueb, vsb, neb, uet, xt, res,
               relT_v, wT_v, b_v, relT_h, wT_h, b_h, out_h, base):
    pltpu.sync_copy(relT_h, relT_v)
    pltpu.sync_copy(wT_h, wT_v)
    pltpu.sync_copy(b_h, b_v)
    lanes = lax.iota(jnp.int32, 16)
    for g in range(NG):
      bvec = lanes + g * 16
      bt, bs = bvec >> 3, bvec & 7
      relids = [plsc.load_gather(arb, [bt, bs, lanes * 0 + k])
                for k in range(K)]
      def logits_step(d, s):
        dvec = lanes * 0 + d
        ued = plsc.load_gather(ueb, [bt, bs, dvec])
        uet[d] = ued
        return tuple(
            s[k] + ued * plsc.load_gather(relT_v, [dvec, relids[k]])
            for k in range(K))
      s = lax.fori_loop(
          0, D, logits_step,
          tuple(jnp.zeros((16,), jnp.float32) for _ in range(K)))
      m = s[0]
      for k in range(1, K):
        m = jnp.maximum(m, s[k])
      es = [jnp.exp(s[k] - m) for k in range(K)]
      tot = es[0]
      for k in range(1, K):
        tot = tot + es[k]
      inv = 1.0 / tot
      p = [es[k] * inv for k in range(K)]
      nrows = [bvec * K + k for k in range(K)]
      nts = [(nrows[k] >> 3, nrows[k] & 7) for k in range(K)]
      def wsum_step(d, carry):
        dvec = lanes * 0 + d
        acc = plsc.load_gather(vsb, [bt, bs, dvec])
        for k in range(K):
          ned = plsc.load_gather(neb, [nts[k][0], nts[k][1], dvec])
          acc = acc + p[k] * ned
        xt[d] = acc
        return carry
      lax.fori_loop(0, D, wsum_step, 0)
      def lin_step(do, y):
        dovec = lanes * 0 + do
        accw = plsc.load_gather(b_v, [dovec])
        for j in range(D):
          wv = plsc.load_gather(wT_v, [lanes * 0 + j, dovec])
          accw = accw + xt[j] * wv
        vu = jnp.maximum(accw, 0.0)
        return y + plsc.load_gather(uet, [dovec, lanes]) * vu
      y = lax.fori_loop(0, D, lin_step, jnp.zeros((16,), jnp.float32))
      res[pl.ds(g * 16, 16)] = 1.0 / (1.0 + jnp.exp(-y))
    pltpu.sync_copy(res, out_h.at[pl.ds(base, BPW)])

  return sc_fused


def kernel(u, v, adj_ent, adj_rel, usr_emb, ent_emb, rel_emb, W, b):
  ae3 = adj_ent.astype(jnp.int32).reshape(NUM_ENT // 8, 8, K)
  ar3 = adj_rel.astype(jnp.int32).reshape(NUM_ENT // 8, 8, K)
  usr3 = usr_emb.reshape(NUM_USR // 8, 8, D)
  ent3 = ent_emb.reshape(NUM_ENT // 8, 8, D)
  return _sc_fused_kernel()(
      u.astype(jnp.int32), v.astype(jnp.int32), ae3, ar3, usr3, ent3,
      rel_emb.T, W.T, b)
